# baseline (speedup 1.0000x reference)
import functools

import jax
import jax.numpy as jnp
from jax import lax
from jax.experimental import pallas as pl
from jax.experimental.pallas import tpu as pltpu
from jax.experimental.pallas import tpu_sc as plsc

_RPW = 8


def _sc_gather_sum_body(n_active, n_cores, x_hbm, emb_hbm, out_hbm,
                        idx_v, rows_v, sum_v, sem):
    wid = lax.axis_index("s") * n_cores + lax.axis_index("c")

    @pl.when(wid < n_active)
    def _():
        pltpu.sync_copy(x_hbm.at[pl.ds(wid * _RPW, _RPW)], idx_v)
        pltpu.async_copy(emb_hbm.at[idx_v], rows_v, sem).wait()
        d = rows_v.shape[1]
        for c in range(d // 16):
            acc = rows_v[0, pl.ds(c * 16, 16)]
            for r in range(1, _RPW):
                acc = acc + rows_v[r, pl.ds(c * 16, 16)]
            sum_v[pl.ds(c * 16, 16)] = acc
        pltpu.sync_copy(sum_v, out_hbm.at[wid])


def _tc_body(K, C, V, L, partials, W1, b1, W2, b2, out, logits_v, h1_v, stat_s):
    i = pl.program_id(0)

    @pl.when(i == 0)
    def _():
        h = jnp.sum(partials[...], axis=0, keepdims=True) * (1.0 / L)
        pre = jnp.dot(h, W1[...], preferred_element_type=jnp.float32) + b1[...]
        h1_v[...] = jnp.maximum(pre, 0.0)
        stat_s[0] = -jnp.inf
        stat_s[1] = 0.0

    @pl.when(i < K)
    def _():
        logits = jnp.dot(h1_v[...], W2[...],
                         preferred_element_type=jnp.float32) + b2[...]
        col = i * C + lax.broadcasted_iota(jnp.int32, (1, C), 1)
        valid = col < V
        m_c = jnp.max(jnp.where(valid, logits, -jnp.inf))
        m_old = stat_s[0]
        m_new = jnp.maximum(m_old, m_c)
        s_new = stat_s[1] * jnp.exp(m_old - m_new) + jnp.sum(
            jnp.where(valid, jnp.exp(logits - m_new), 0.0))
        stat_s[0] = m_new
        stat_s[1] = s_new
        logits_v[:, pl.ds(i * C, C)] = logits

        @pl.when(i == K - 1)
        def _():
            stat_s[2] = m_new + jnp.log(s_new)

    @pl.when(i >= K)
    def _():
        j = i - K
        out[...] = logits_v[:, pl.ds(j * C, C)] - stat_s[2]


def _gather_pool_sc(x, emb, n_active):
    info = plsc.get_sparse_core_info()
    n_cores = info.num_cores
    d = emb.shape[1]
    mesh = plsc.VectorSubcoreMesh(core_axis_name="c", subcore_axis_name="s")
    body = functools.partial(_sc_gather_sum_body, n_active, n_cores)
    call = pl.kernel(
        body,
        mesh=mesh,
        out_type=jax.ShapeDtypeStruct((n_active, d), jnp.float32),
        scratch_types=[
            pltpu.VMEM((_RPW,), jnp.int32),
            pltpu.VMEM((_RPW, d), jnp.float32),
            pltpu.VMEM((d,), jnp.float32),
            pltpu.SemaphoreType.DMA,
        ],
    )
    return call(x, emb)


def kernel(x, emb, W1, b1, W2, b2):
    x = x.astype(jnp.int32)
    L = x.shape[0]
    D = emb.shape[1]
    H = W1.shape[1]
    V = W2.shape[1]
    n_active = L // _RPW

    partials = _gather_pool_sc(x, emb, n_active)

    C = 8192
    K = -(-V // C)
    b1v = b1.reshape(1, H)
    b2v = b2.reshape(1, V)

    out = pl.pallas_call(
        functools.partial(_tc_body, K, C, V, L),
        grid=(2 * K,),
        in_specs=[
            pl.BlockSpec((n_active, D), lambda i: (0, 0)),
            pl.BlockSpec((D, H), lambda i: (0, 0)),
            pl.BlockSpec((1, H), lambda i: (0, 0)),
            pl.BlockSpec((H, C), lambda i: (0, jnp.minimum(i, K - 1))),
            pl.BlockSpec((1, C), lambda i: (0, jnp.minimum(i, K - 1))),
        ],
        out_specs=pl.BlockSpec((1, C), lambda i: (0, jnp.where(i < K, 0, i - K))),
        out_shape=jax.ShapeDtypeStruct((1, V), jnp.float32),
        scratch_shapes=[
            pltpu.VMEM((1, K * C), jnp.float32),
            pltpu.VMEM((1, H), jnp.float32),
            pltpu.SMEM((3,), jnp.float32),
        ],
    )(partials, W1, b1v, W2, b2v)
    return out


# speedup vs baseline: 1.2015x; 1.2015x over previous
import functools

import jax
import jax.numpy as jnp
from jax import lax
from jax.experimental import pallas as pl
from jax.experimental.pallas import tpu as pltpu
from jax.experimental.pallas import tpu_sc as plsc

_RPW = 8


def _sc_gather_sum_body(n_active, n_cores, x_hbm, emb_hbm, out_hbm,
                        idx_v, rows_v, sum_v, sem):
    wid = lax.axis_index("s") * n_cores + lax.axis_index("c")

    @pl.when(wid < n_active)
    def _():
        pltpu.sync_copy(x_hbm.at[pl.ds(wid * _RPW, _RPW)], idx_v)
        pltpu.async_copy(emb_hbm.at[idx_v], rows_v, sem).wait()
        d = rows_v.shape[1]
        for c in range(d // 16):
            acc = rows_v[0, pl.ds(c * 16, 16)]
            for r in range(1, _RPW):
                acc = acc + rows_v[r, pl.ds(c * 16, 16)]
            sum_v[pl.ds(c * 16, 16)] = acc
        pltpu.sync_copy(sum_v, out_hbm.at[wid])


def _tc_body(NS, KS, C, C_OUT, V, L, *refs):
    partials, W1, b1 = refs[0], refs[1], refs[2]
    w2s = refs[3:3 + NS]
    b2s = refs[3 + NS:3 + 2 * NS]
    out, logits_v, h1_v, stat_s = refs[3 + 2 * NS:]
    i = pl.program_id(0)

    @pl.when(i == 0)
    def _():
        h = jnp.sum(partials[...], axis=0, keepdims=True) * (1.0 / L)
        pre = jnp.dot(h, W1[...], preferred_element_type=jnp.float32) + b1[...]
        h1_v[...] = jnp.maximum(pre, 0.0)
        stat_s[0] = -jnp.inf
        stat_s[1] = 0.0

    @pl.when(i < KS)
    def _():
        h1 = h1_v[...]
        m_run = stat_s[0]
        s_run = stat_s[1]
        for s in range(NS):
            logits = jnp.dot(h1, w2s[s][...],
                             preferred_element_type=jnp.float32) + b2s[s][...]
            if s == NS - 1:
                col = (s * KS + i) * C + lax.broadcasted_iota(jnp.int32, (1, C), 1)
                lmask = jnp.where(col < V, logits, -jnp.inf)
            else:
                lmask = logits
            m_new = jnp.maximum(m_run, jnp.max(lmask))
            s_run = s_run * jnp.exp(m_run - m_new) + jnp.sum(jnp.exp(lmask - m_new))
            m_run = m_new
            logits_v[:, pl.ds((s * KS + i) * C, C)] = logits
        stat_s[0] = m_run
        stat_s[1] = s_run

        @pl.when(i == KS - 1)
        def _():
            stat_s[2] = m_run + jnp.log(s_run)

    @pl.when(i >= KS)
    def _():
        j = i - KS
        out[...] = logits_v[:, pl.ds(j * C_OUT, C_OUT)] - stat_s[2]


def _gather_pool_sc(x, emb, n_active):
    info = plsc.get_sparse_core_info()
    n_cores = info.num_cores
    d = emb.shape[1]
    mesh = plsc.VectorSubcoreMesh(core_axis_name="c", subcore_axis_name="s")
    body = functools.partial(_sc_gather_sum_body, n_active, n_cores)
    call = pl.kernel(
        body,
        mesh=mesh,
        out_type=jax.ShapeDtypeStruct((n_active, d), jnp.float32),
        scratch_types=[
            pltpu.VMEM((_RPW,), jnp.int32),
            pltpu.VMEM((_RPW, d), jnp.float32),
            pltpu.VMEM((d,), jnp.float32),
            pltpu.SemaphoreType.DMA,
        ],
    )
    return call(x, emb)


def kernel(x, emb, W1, b1, W2, b2):
    x = x.astype(jnp.int32)
    L = x.shape[0]
    D = emb.shape[1]
    H = W1.shape[1]
    V = W2.shape[1]
    n_active = L // _RPW

    partials = _gather_pool_sc(x, emb, n_active)

    NS = 4
    C = 6400
    KS = 4
    C_OUT = 25600
    K_OUT = -(-V // C_OUT)
    b1v = b1.reshape(1, H)
    b2v = b2.reshape(1, V)

    def _w2_spec(s):
        return pl.BlockSpec((H, C), lambda i: (0, s * KS + jnp.minimum(i, KS - 1)))

    def _b2_spec(s):
        return pl.BlockSpec((1, C), lambda i: (0, s * KS + jnp.minimum(i, KS - 1)))

    out = pl.pallas_call(
        functools.partial(_tc_body, NS, KS, C, C_OUT, V, L),
        grid=(KS + K_OUT,),
        in_specs=[
            pl.BlockSpec((n_active, D), lambda i: (0, 0)),
            pl.BlockSpec((D, H), lambda i: (0, 0)),
            pl.BlockSpec((1, H), lambda i: (0, 0)),
        ] + [_w2_spec(s) for s in range(NS)] + [_b2_spec(s) for s in range(NS)],
        out_specs=pl.BlockSpec(
            (1, C_OUT), lambda i: (0, jnp.where(i < KS, 0, i - KS))),
        out_shape=jax.ShapeDtypeStruct((1, V), jnp.float32),
        scratch_shapes=[
            pltpu.VMEM((1, NS * KS * C), jnp.float32),
            pltpu.VMEM((1, H), jnp.float32),
            pltpu.SMEM((3,), jnp.float32),
        ],
    )(partials, W1, b1v, *([W2] * NS), *([b2v] * NS))
    return out
